# baseline (speedup 1.0000x reference)
import functools

import jax
import jax.numpy as jnp
from jax import lax
from jax.experimental import pallas as pl
from jax.experimental.pallas import tpu as pltpu
from jax.experimental.pallas import tpu_sc as plsc

N_NODES = 10000
N_PAD = 10112
DUMMY = 10008
NC = 2
NS = 16
NW = NC * NS
EB = 128
STRIPE = N_PAD // NS
H = 16
DH = 64
GD = H * DH
NCHUNK = 16
CW = GD // NCHUNK


def _pad_edges(s, d, mult):
    e = s.shape[0]
    per = -(-e // (NW * mult)) * mult
    pad = per * NW - e
    s = jnp.concatenate([s, jnp.zeros((pad,), jnp.int32)])
    d = jnp.concatenate([d, jnp.full((pad,), DUMMY, jnp.int32)])
    return s, d, per


_MESH = plsc.VectorSubcoreMesh(core_axis_name="c", subcore_axis_name="s")
_SC_PARAMS = pltpu.CompilerParams(use_tc_tiling_on_sc=False,
                                  needs_layout_passes=False)


def _seg_sum_rows(table3d, srcp, dstp, per_tile):
    nb = per_tile // EB
    P, _, width = table3d.shape

    @functools.partial(
        pl.kernel,
        out_type=jax.ShapeDtypeStruct((P, NC, N_PAD, width), jnp.float32),
        mesh=_MESH,
        scratch_types=[
            pltpu.VMEM((EB,), jnp.int32),
            pltpu.VMEM((EB,), jnp.int32),
            pltpu.VMEM((EB, width), jnp.float32),
            pltpu.VMEM_SHARED((N_PAD, width), jnp.float32),
            pltpu.SemaphoreType.DMA,
        ],
        compiler_params=_SC_PARAMS,
    )
    def k(table_hbm, src_hbm, dst_hbm, zeros_hbm, out_hbm, sidx, didx, rows,
          acc, sem):
        c = lax.axis_index("c")
        s = lax.axis_index("s")
        wid = s * NC + c
        base = wid * per_tile
        for p in range(P):
            pltpu.sync_copy(zeros_hbm.at[pl.ds(s * STRIPE, STRIPE)],
                            acc.at[pl.ds(s * STRIPE, STRIPE)])
            plsc.subcore_barrier()

            def body(i, _):
                off = base + i * EB
                pltpu.sync_copy(src_hbm.at[pl.ds(off, EB)], sidx)
                pltpu.sync_copy(dst_hbm.at[pl.ds(off, EB)], didx)
                pltpu.async_copy(table_hbm.at[p].at[sidx], rows, sem).wait()
                pltpu.sync_copy(rows, acc.at[didx], add=True)
                return 0

            lax.fori_loop(0, nb, body, 0)
            plsc.subcore_barrier()
            pltpu.sync_copy(acc.at[pl.ds(s * STRIPE, STRIPE)],
                            out_hbm.at[p].at[c].at[pl.ds(s * STRIPE, STRIPE)])
            plsc.subcore_barrier()

    zeros = jnp.zeros((N_PAD, width), jnp.float32)
    return k(table3d, srcp, dstp, zeros)


def _edge_softmax(AS, BC, sE, dE, per_tile):
    nb = per_tile // EB
    ep = per_tile * NW

    @functools.partial(
        pl.kernel,
        out_type=[
            jax.ShapeDtypeStruct((ep, H), jnp.float32),
            jax.ShapeDtypeStruct((NC, N_PAD, H), jnp.float32),
        ],
        mesh=_MESH,
        scratch_types=[
            pltpu.VMEM((EB,), jnp.int32),
            pltpu.VMEM((EB,), jnp.int32),
            pltpu.VMEM((EB, H), jnp.float32),
            pltpu.VMEM((EB, 2 * H), jnp.float32),
            pltpu.VMEM((EB, H), jnp.float32),
            pltpu.VMEM_SHARED((N_PAD, H), jnp.float32),
            pltpu.SemaphoreType.DMA,
            pltpu.SemaphoreType.DMA,
        ],
        compiler_params=_SC_PARAMS,
    )
    def k(as_hbm, bc_hbm, s_hbm, d_hbm, zeros_hbm, w_hbm, den_hbm,
          sidx, didx, asb, bcb, wbuf, acc, sem1, sem2):
        c = lax.axis_index("c")
        s = lax.axis_index("s")
        wid = s * NC + c
        pltpu.sync_copy(zeros_hbm.at[pl.ds(s * STRIPE, STRIPE)],
                        acc.at[pl.ds(s * STRIPE, STRIPE)])
        plsc.subcore_barrier()
        base = wid * per_tile

        def body(i, _):
            off = base + i * EB
            pltpu.sync_copy(s_hbm.at[pl.ds(off, EB)], sidx)
            pltpu.sync_copy(d_hbm.at[pl.ds(off, EB)], didx)
            ca = pltpu.async_copy(as_hbm.at[sidx], asb, sem1)
            cb = pltpu.async_copy(bc_hbm.at[didx], bcb, sem2)
            ca.wait()
            cb.wait()

            def edge(j, _):
                t = asb[j] + bcb[j, pl.ds(0, H)]
                e = jnp.maximum(t, 0.2 * t)
                wbuf[j] = jnp.exp(e - bcb[j, pl.ds(H, H)])
                return 0

            lax.fori_loop(0, EB, edge, 0)
            pltpu.sync_copy(wbuf, acc.at[didx], add=True)
            pltpu.sync_copy(wbuf, w_hbm.at[pl.ds(off, EB)])
            return 0

        lax.fori_loop(0, nb, body, 0)
        plsc.subcore_barrier()
        pltpu.sync_copy(acc.at[pl.ds(s * STRIPE, STRIPE)],
                        den_hbm.at[c].at[pl.ds(s * STRIPE, STRIPE)])

    zeros = jnp.zeros((N_PAD, H), jnp.float32)
    return k(AS, BC, sE, dE, zeros)


def _gat_messages(g_cm, w, sE, dE, ID, b_gat, per_tile):
    nb = per_tile // EB
    rb = STRIPE // 4
    CPC = NCHUNK // NC

    @functools.partial(
        pl.kernel,
        out_type=jax.ShapeDtypeStruct((NCHUNK, N_PAD, CW), jnp.float32),
        mesh=_MESH,
        scratch_types=[
            pltpu.VMEM((nb, EB), jnp.int32),
            pltpu.VMEM((nb, EB), jnp.int32),
            pltpu.VMEM((3, EB, H), jnp.float32),
            pltpu.VMEM((3, EB, CW), jnp.float32),
            pltpu.VMEM((rb, CW), jnp.float32),
            pltpu.VMEM((rb, H), jnp.float32),
            pltpu.VMEM((CW,), jnp.float32),
            pltpu.VMEM_SHARED((N_PAD, CW), jnp.float32),
            pltpu.SemaphoreType.DMA((3,)),
            pltpu.SemaphoreType.DMA((3,)),
            pltpu.SemaphoreType.DMA((3,)),
        ],
        compiler_params=_SC_PARAMS,
    )
    def k(g_hbm, w_hbm, s2_hbm, d2_hbm, id_hbm, bg_hbm, zeros_hbm, out_hbm,
          sidx, didx, wring, gring, vbuf, idb, bg, acc, gsem, wsem, ssem):
        c = lax.axis_index("c")
        s = lax.axis_index("s")
        pltpu.sync_copy(s2_hbm.at[pl.ds(s * nb, nb)], sidx)
        pltpu.sync_copy(d2_hbm.at[pl.ds(s * nb, nb)], didx)
        ebase = s * per_tile

        def gather(ck, i, slot):
            pltpu.make_async_copy(
                g_hbm.at[ck].at[sidx.at[i]], gring.at[slot],
                gsem.at[slot]).start()
            pltpu.make_async_copy(
                w_hbm.at[pl.ds(ebase + i * EB, EB)], wring.at[slot],
                wsem.at[slot]).start()

        def wait_gather(ck, i, slot):
            pltpu.make_async_copy(
                g_hbm.at[ck].at[sidx.at[i]], gring.at[slot],
                gsem.at[slot]).wait()
            pltpu.make_async_copy(
                w_hbm.at[pl.ds(ebase + i * EB, EB)], wring.at[slot],
                wsem.at[slot]).wait()

        def wait_scatter(i, slot):
            pltpu.make_async_copy(
                gring.at[slot], acc.at[didx.at[i]], ssem.at[slot]).wait()

        def do_chunk(kk, _):
            ck = c * CPC + kk
            pltpu.sync_copy(zeros_hbm.at[pl.ds(s * STRIPE, STRIPE)],
                            acc.at[pl.ds(s * STRIPE, STRIPE)])
            pltpu.sync_copy(bg_hbm.at[pl.ds(ck * CW, CW)], bg)
            plsc.subcore_barrier()
            gather(ck, 0, 0)
            gather(ck, 1, 1)
            ckv = jnp.full((16,), ck, jnp.int32)

            def compute(slot):
                slotv = jnp.full((16,), slot, jnp.int32)
                for q in range(EB // 16):
                    wq = plsc.load_gather(
                        wring, [slotv, lax.iota(jnp.int32, 16) + 16 * q,
                                ckv])
                    for l in range(16):
                        j = 16 * q + l
                        a = jnp.full((16,), wq[l])
                        for z in range(CW // 16):
                            gring[slot, j, pl.ds(z * 16, 16)] = (
                                gring[slot, j, pl.ds(z * 16, 16)] * a)

            def body(ii, _):
                for sub in range(3):
                    i = 3 * ii + sub
                    nxt = (sub + 2) % 3

                    @pl.when(i >= 1)
                    def _():
                        wait_scatter(i - 1, nxt)

                    @pl.when(i + 2 < nb)
                    def _():
                        gather(ck, i + 2, nxt)

                    wait_gather(ck, i, sub)
                    compute(sub)
                    pltpu.async_copy(gring.at[sub], acc.at[didx.at[i]],
                                     ssem.at[sub], add=True)
                return 0

            lax.fori_loop(0, nb // 3, body, 0)
            wait_scatter(nb - 1, (nb - 1) % 3)
            plsc.subcore_barrier()
            ckv = jnp.full((16,), ck, jnp.int32)
            for q in range(4):
                r0 = s * STRIPE + q * rb
                pltpu.sync_copy(acc.at[pl.ds(r0, rb)], vbuf)
                pltpu.sync_copy(id_hbm.at[pl.ds(r0, rb)], idb)

                def row(r, _):
                    i0 = plsc.load_gather(
                        idb, [jnp.full((16,), r, jnp.int32), ckv])
                    for z in range(CW // 16):
                        vbuf[r, pl.ds(z * 16, 16)] = jnp.maximum(
                            vbuf[r, pl.ds(z * 16, 16)] * i0
                            + bg[pl.ds(z * 16, 16)], 0.0)
                    return 0

                lax.fori_loop(0, rb, row, 0)
                pltpu.sync_copy(vbuf, out_hbm.at[ck].at[pl.ds(r0, rb)])
            plsc.subcore_barrier()
            return 0

        lax.fori_loop(0, CPC, do_chunk, 0)

    zeros = jnp.zeros((N_PAD, CW), jnp.float32)
    s2 = sE.reshape(-1, EB)
    d2 = dE.reshape(-1, EB)
    return k(g_cm, w, s2, d2, ID, b_gat, zeros)


def _tc1_body(x_ref, w_ref, b_ref, xr_ref, xroot_ref):
    xw = x_ref[...] @ w_ref[...]
    xr_ref[0] = xw[:, :64]
    xroot_ref[...] = xw[:, 64:] + b_ref[...]


def _tc2a_body(p_ref, xroot_ref, h_ref):
    h_ref[...] = jnp.maximum(
        p_ref[0, 0, :N_NODES] + p_ref[0, 1, :N_NODES] + xroot_ref[...], 0.0)


def _tc2b_body(h_ref, w_ref, out_ref):
    out_ref[0] = h_ref[...] @ w_ref[0]


def _tc2c_body(h_ref, wg_ref, asrc_ref, adst_ref, as_ref, bc_ref):
    wg = wg_ref[...].reshape(DH, H, DH)
    A = jnp.concatenate(
        [(wg * asrc_ref[...][None]).sum(-1),
         (wg * adst_ref[...][None]).sum(-1)], axis=1)
    ab = h_ref[...] @ A
    a_s = ab[:, :H]
    a_d = ab[:, H:]
    mhat_t = a_d + jnp.max(a_s)
    mhat = jnp.maximum(mhat_t, 0.2 * mhat_t)
    as_ref[...] = a_s
    bc = jnp.concatenate([a_d, mhat], axis=1)
    bc_ref[...] = jnp.concatenate(
        [bc, jnp.zeros((N_PAD - N_NODES, 2 * H), jnp.float32)], axis=0)


def _tc3_body(den_ref, id_ref):
    id_ref[...] = 1.0 / (den_ref[0] + den_ref[1] + 1e-16)


def _tc4_body(h2_ref, w_ref, out_ref):
    c = pl.program_id(0)

    @pl.when(c == 0)
    def _():
        out_ref[...] = jnp.zeros_like(out_ref)

    out_ref[...] += h2_ref[0, :N_NODES] @ w_ref[0]


def _tc6_body(p2_ref, y2r2_ref, b5_ref, batch_ref, wfc1_ref, bfc1_ref,
              wfc2_ref, bfc2_ref, out_ref):
    agg2 = jnp.concatenate(
        [p2_ref[0, 0, :N_NODES] + p2_ref[0, 1, :N_NODES],
         p2_ref[1, 0, :N_NODES] + p2_ref[1, 1, :N_NODES]], axis=1)
    h3 = jnp.maximum(agg2 + y2r2_ref[:, 64:] + b5_ref[...], 0.0)
    seg = jax.lax.broadcasted_iota(jnp.int32, (64, N_NODES), 0)
    mask = (seg == batch_ref[...][None, :]).astype(jnp.float32)
    pooled = mask @ h3
    z = jnp.maximum(pooled @ wfc1_ref[...] + bfc1_ref[...], 0.0)
    logits = z @ wfc2_ref[...] + bfc2_ref[...]
    out_ref[...] = jax.nn.sigmoid(logits)


def kernel(x, edge_index, batch, W_rel1, W_root1, b1, W_gat, att_src, att_dst,
           b_gat, W_rel5, W_root5, b5, W_fc1, b_fc1, W_fc2, b_fc2):
    n = x.shape[0]
    src = edge_index[0]
    dst = edge_index[1]

    xr, xroot = pl.pallas_call(
        _tc1_body,
        out_shape=[jax.ShapeDtypeStruct((1, n, 64), jnp.float32),
                   jax.ShapeDtypeStruct((n, 64), jnp.float32)],
    )(x, jnp.concatenate([W_rel1, W_root1], axis=1), b1)
    srcp, dstp, per = _pad_edges(src, dst, EB)
    parts = _seg_sum_rows(xr, srcp, dstp, per)
    h = pl.pallas_call(
        _tc2a_body,
        out_shape=jax.ShapeDtypeStruct((n, 64), jnp.float32),
    )(parts, xroot)

    loop = jnp.arange(n, dtype=src.dtype)
    sE, dE, perE = _pad_edges(jnp.concatenate([src, loop]),
                              jnp.concatenate([dst, loop]), EB)
    g_cm = pl.pallas_call(
        _tc2b_body,
        grid=(NCHUNK,),
        in_specs=[pl.BlockSpec((n, 64), lambda c: (0, 0)),
                  pl.BlockSpec((1, 64, CW), lambda c: (c, 0, 0))],
        out_specs=pl.BlockSpec((1, n, CW), lambda c: (c, 0, 0)),
        out_shape=jax.ShapeDtypeStruct((NCHUNK, n, CW), jnp.float32),
    )(h, jnp.moveaxis(W_gat.reshape(64, NCHUNK, CW), 1, 0))
    AS, BC = pl.pallas_call(
        _tc2c_body,
        out_shape=[jax.ShapeDtypeStruct((n, H), jnp.float32),
                   jax.ShapeDtypeStruct((N_PAD, 2 * H), jnp.float32)],
    )(h, W_gat, att_src, att_dst)
    w, denp = _edge_softmax(AS, BC, sE, dE, perE)
    ID = pl.pallas_call(
        _tc3_body,
        out_shape=jax.ShapeDtypeStruct((N_PAD, H), jnp.float32),
    )(denp)
    h2_cm = _gat_messages(g_cm, w, sE, dE, ID, b_gat, perE * NC)

    y2r2 = pl.pallas_call(
        _tc4_body,
        grid=(NCHUNK,),
        in_specs=[pl.BlockSpec((1, N_PAD, CW), lambda c: (c, 0, 0)),
                  pl.BlockSpec((1, CW, 128), lambda c: (c, 0, 0))],
        out_specs=pl.BlockSpec((n, 128), lambda c: (0, 0)),
        out_shape=jax.ShapeDtypeStruct((n, 128), jnp.float32),
    )(h2_cm, jnp.concatenate([W_rel5, W_root5], axis=1).reshape(
        NCHUNK, CW, 128))
    y232 = jnp.stack([y2r2[:, :32], y2r2[:, 32:64]])
    parts2 = _seg_sum_rows(y232, srcp, dstp, per)
    return pl.pallas_call(
        _tc6_body,
        out_shape=jax.ShapeDtypeStruct((64, W_fc2.shape[1]), jnp.float32),
    )(parts2, y2r2, b5, batch, W_fc1, b_fc1, W_fc2, b_fc2)


# speedup vs baseline: 1.0055x; 1.0055x over previous
import functools

import jax
import jax.numpy as jnp
from jax import lax
from jax.experimental import pallas as pl
from jax.experimental.pallas import tpu as pltpu
from jax.experimental.pallas import tpu_sc as plsc

N_NODES = 10000
N_PAD = 10112
DUMMY = 10008
NC = 2
NS = 16
NW = NC * NS
EB = 128
STRIPE = N_PAD // NS
H = 16
DH = 64
GD = H * DH
NCHUNK = 16
CW = GD // NCHUNK


def _pad_edges(s, d, mult):
    e = s.shape[0]
    per = -(-e // (NW * mult)) * mult
    pad = per * NW - e
    s = jnp.concatenate([s, jnp.zeros((pad,), jnp.int32)])
    d = jnp.concatenate([d, jnp.full((pad,), DUMMY, jnp.int32)])
    return s, d, per


_MESH = plsc.VectorSubcoreMesh(core_axis_name="c", subcore_axis_name="s")
_SC_PARAMS = pltpu.CompilerParams(use_tc_tiling_on_sc=False,
                                  needs_layout_passes=False)


def _seg_sum_rows(table3d, srcp, dstp, per_tile):
    nb = per_tile // EB
    P, _, width = table3d.shape

    @functools.partial(
        pl.kernel,
        out_type=jax.ShapeDtypeStruct((P, NC, N_PAD, width), jnp.float32),
        mesh=_MESH,
        scratch_types=[
            pltpu.VMEM((EB,), jnp.int32),
            pltpu.VMEM((EB,), jnp.int32),
            pltpu.VMEM((EB, width), jnp.float32),
            pltpu.VMEM_SHARED((N_PAD, width), jnp.float32),
            pltpu.SemaphoreType.DMA,
        ],
        compiler_params=_SC_PARAMS,
    )
    def k(table_hbm, src_hbm, dst_hbm, zeros_hbm, out_hbm, sidx, didx, rows,
          acc, sem):
        c = lax.axis_index("c")
        s = lax.axis_index("s")
        wid = s * NC + c
        base = wid * per_tile
        for p in range(P):
            pltpu.sync_copy(zeros_hbm.at[pl.ds(s * STRIPE, STRIPE)],
                            acc.at[pl.ds(s * STRIPE, STRIPE)])
            plsc.subcore_barrier()

            def body(i, _):
                off = base + i * EB
                pltpu.sync_copy(src_hbm.at[pl.ds(off, EB)], sidx)
                pltpu.sync_copy(dst_hbm.at[pl.ds(off, EB)], didx)
                pltpu.async_copy(table_hbm.at[p].at[sidx], rows, sem).wait()
                pltpu.sync_copy(rows, acc.at[didx], add=True)
                return 0

            lax.fori_loop(0, nb, body, 0)
            plsc.subcore_barrier()
            pltpu.sync_copy(acc.at[pl.ds(s * STRIPE, STRIPE)],
                            out_hbm.at[p].at[c].at[pl.ds(s * STRIPE, STRIPE)])
            plsc.subcore_barrier()

    zeros = jnp.zeros((N_PAD, width), jnp.float32)
    return k(table3d, srcp, dstp, zeros)


def _edge_softmax(AS, BC, sE, dE, per_tile):
    nb = per_tile // EB
    ep = per_tile * NW

    @functools.partial(
        pl.kernel,
        out_type=jax.ShapeDtypeStruct((ep, H), jnp.float32),
        mesh=_MESH,
        scratch_types=[
            pltpu.VMEM((EB,), jnp.int32),
            pltpu.VMEM((EB,), jnp.int32),
            pltpu.VMEM((EB, H), jnp.float32),
            pltpu.VMEM((EB, 2 * H), jnp.float32),
            pltpu.VMEM((EB, H), jnp.float32),
            pltpu.SemaphoreType.DMA,
            pltpu.SemaphoreType.DMA,
        ],
        compiler_params=_SC_PARAMS,
    )
    def k(as_hbm, bc_hbm, s_hbm, d_hbm, w_hbm,
          sidx, didx, asb, bcb, wbuf, sem1, sem2):
        c = lax.axis_index("c")
        s = lax.axis_index("s")
        wid = s * NC + c
        base = wid * per_tile

        def body(i, _):
            off = base + i * EB
            pltpu.sync_copy(s_hbm.at[pl.ds(off, EB)], sidx)
            pltpu.sync_copy(d_hbm.at[pl.ds(off, EB)], didx)
            ca = pltpu.async_copy(as_hbm.at[sidx], asb, sem1)
            cb = pltpu.async_copy(bc_hbm.at[didx], bcb, sem2)
            ca.wait()
            cb.wait()

            def edge(j, _):
                t = asb[j] + bcb[j, pl.ds(0, H)]
                e = jnp.maximum(t, 0.2 * t)
                wbuf[j] = jnp.exp(e - bcb[j, pl.ds(H, H)])
                return 0

            lax.fori_loop(0, EB, edge, 0)
            pltpu.sync_copy(wbuf, w_hbm.at[pl.ds(off, EB)])
            return 0

        lax.fori_loop(0, nb, body, 0)

    return k(AS, BC, sE, dE)


def _gat_messages(g_cm, w, sE, dE, b_gat, per_tile):
    nb = per_tile // EB
    rb = STRIPE // 4
    CPC = NCHUNK // NC

    @functools.partial(
        pl.kernel,
        out_type=jax.ShapeDtypeStruct((NCHUNK, N_PAD, CW), jnp.float32),
        mesh=_MESH,
        scratch_types=[
            pltpu.VMEM((nb, EB), jnp.int32),
            pltpu.VMEM((nb, EB), jnp.int32),
            pltpu.VMEM((3, EB, H), jnp.float32),
            pltpu.VMEM((3, EB, CW), jnp.float32),
            pltpu.VMEM((rb, CW), jnp.float32),
            pltpu.VMEM((rb, H), jnp.float32),
            pltpu.VMEM((CW,), jnp.float32),
            pltpu.VMEM_SHARED((N_PAD, CW), jnp.float32),
            pltpu.VMEM_SHARED((N_PAD, H), jnp.float32),
            pltpu.SemaphoreType.DMA((3,)),
            pltpu.SemaphoreType.DMA((3,)),
            pltpu.SemaphoreType.DMA((3,)),
            pltpu.SemaphoreType.DMA((3,)),
        ],
        compiler_params=_SC_PARAMS,
    )
    def k(g_hbm, w_hbm, s2_hbm, d2_hbm, bg_hbm, zeros_hbm, zeros16_hbm,
          out_hbm, sidx, didx, wring, gring, vbuf, idb, bg, acc, den,
          gsem, wsem, ssem, dsem):
        c = lax.axis_index("c")
        s = lax.axis_index("s")
        pltpu.sync_copy(s2_hbm.at[pl.ds(s * nb, nb)], sidx)
        pltpu.sync_copy(d2_hbm.at[pl.ds(s * nb, nb)], didx)
        ebase = s * per_tile

        def gather(ck, i, slot):
            pltpu.make_async_copy(
                g_hbm.at[ck].at[sidx.at[i]], gring.at[slot],
                gsem.at[slot]).start()
            pltpu.make_async_copy(
                w_hbm.at[pl.ds(ebase + i * EB, EB)], wring.at[slot],
                wsem.at[slot]).start()

        def wait_gather(ck, i, slot):
            pltpu.make_async_copy(
                g_hbm.at[ck].at[sidx.at[i]], gring.at[slot],
                gsem.at[slot]).wait()
            pltpu.make_async_copy(
                w_hbm.at[pl.ds(ebase + i * EB, EB)], wring.at[slot],
                wsem.at[slot]).wait()

        def wait_scatter(i, slot):
            pltpu.make_async_copy(
                gring.at[slot], acc.at[didx.at[i]], ssem.at[slot]).wait()

        def do_chunk(kk, _):
            ck = c * CPC + kk
            pltpu.sync_copy(zeros_hbm.at[pl.ds(s * STRIPE, STRIPE)],
                            acc.at[pl.ds(s * STRIPE, STRIPE)])

            @pl.when(kk == 0)
            def _():
                pltpu.sync_copy(zeros16_hbm.at[pl.ds(s * STRIPE, STRIPE)],
                                den.at[pl.ds(s * STRIPE, STRIPE)])

            pltpu.sync_copy(bg_hbm.at[pl.ds(ck * CW, CW)], bg)
            plsc.subcore_barrier()
            gather(ck, 0, 0)
            gather(ck, 1, 1)
            c0v = jnp.full((16,), ck, jnp.int32)

            def compute(slot):
                slotv = jnp.full((16,), slot, jnp.int32)
                for q in range(EB // 16):
                    rows = lax.iota(jnp.int32, 16) + 16 * q
                    wq0 = plsc.load_gather(wring, [slotv, rows, c0v])
                    for l in range(16):
                        j = 16 * q + l
                        a0 = jnp.full((16,), wq0[l])
                        for z in range(CW // 16):
                            gring[slot, j, pl.ds(z * 16, 16)] = (
                                gring[slot, j, pl.ds(z * 16, 16)] * a0)

            def body(ii, _):
                for sub in range(3):
                    i = 3 * ii + sub
                    nxt = (sub + 2) % 3

                    @pl.when(i >= 1)
                    def _():
                        wait_scatter(i - 1, nxt)

                    @pl.when(i + 2 < nb)
                    def _():
                        gather(ck, i + 2, nxt)

                    wait_gather(ck, i, sub)

                    @pl.when(kk == 0)
                    def _():
                        pltpu.async_copy(wring.at[sub],
                                         den.at[didx.at[i]],
                                         dsem.at[sub], add=True)

                    compute(sub)

                    @pl.when(kk == 0)
                    def _():
                        pltpu.make_async_copy(wring.at[sub],
                                              den.at[didx.at[i]],
                                              dsem.at[sub]).wait()

                    pltpu.async_copy(gring.at[sub], acc.at[didx.at[i]],
                                     ssem.at[sub], add=True)
                return 0

            lax.fori_loop(0, nb // 3, body, 0)
            wait_scatter(nb - 1, (nb - 1) % 3)
            plsc.subcore_barrier()
            for q in range(4):
                r0 = s * STRIPE + q * rb
                pltpu.sync_copy(acc.at[pl.ds(r0, rb)], vbuf)
                pltpu.sync_copy(den.at[pl.ds(r0, rb)], idb)

                def row(r, _):
                    rv = jnp.full((16,), r, jnp.int32)
                    d0 = plsc.load_gather(idb, [rv, c0v])
                    i0 = 1.0 / (d0 + 1e-16)
                    for z in range(CW // 16):
                        vbuf[r, pl.ds(z * 16, 16)] = jnp.maximum(
                            vbuf[r, pl.ds(z * 16, 16)] * i0
                            + bg[pl.ds(z * 16, 16)], 0.0)
                    return 0

                lax.fori_loop(0, rb, row, 0)
                pltpu.sync_copy(vbuf, out_hbm.at[ck].at[pl.ds(r0, rb)])
            plsc.subcore_barrier()
            return 0

        lax.fori_loop(0, CPC, do_chunk, 0)

    zeros = jnp.zeros((N_PAD, CW), jnp.float32)
    zeros16 = jnp.zeros((N_PAD, H), jnp.float32)
    s2 = sE.reshape(-1, EB)
    d2 = dE.reshape(-1, EB)
    return k(g_cm, w, s2, d2, b_gat, zeros, zeros16)


def _tc1_body(x_ref, w_ref, b_ref, xr_ref, xroot_ref):
    xw = x_ref[...] @ w_ref[...]
    xr_ref[0] = xw[:, :64]
    xroot_ref[...] = xw[:, 64:] + b_ref[...]


def _tc2a_body(p_ref, xroot_ref, h_ref):
    h_ref[...] = jnp.maximum(
        p_ref[0, 0, :N_NODES] + p_ref[0, 1, :N_NODES] + xroot_ref[...], 0.0)


def _tc2b_body(h_ref, w_ref, out_ref):
    out_ref[0] = h_ref[...] @ w_ref[0]


def _tc2c_body(h_ref, wg_ref, asrc_ref, adst_ref, as_ref, bc_ref):
    wg = wg_ref[...].reshape(DH, H, DH)
    A = jnp.concatenate(
        [(wg * asrc_ref[...][None]).sum(-1),
         (wg * adst_ref[...][None]).sum(-1)], axis=1)
    ab = h_ref[...] @ A
    a_s = ab[:, :H]
    a_d = ab[:, H:]
    mhat_t = a_d + jnp.max(a_s)
    mhat = jnp.maximum(mhat_t, 0.2 * mhat_t)
    as_ref[...] = a_s
    bc = jnp.concatenate([a_d, mhat], axis=1)
    bc_ref[...] = jnp.concatenate(
        [bc, jnp.zeros((N_PAD - N_NODES, 2 * H), jnp.float32)], axis=0)


def _tc4_body(h2_ref, w_ref, out_ref):
    c = pl.program_id(0)

    @pl.when(c == 0)
    def _():
        out_ref[...] = jnp.zeros_like(out_ref)

    out_ref[...] += h2_ref[0, :N_NODES] @ w_ref[0]


def _tc6_body(p2_ref, y2r2_ref, b5_ref, batch_ref, wfc1_ref, bfc1_ref,
              wfc2_ref, bfc2_ref, out_ref):
    agg2 = jnp.concatenate(
        [p2_ref[0, 0, :N_NODES] + p2_ref[0, 1, :N_NODES],
         p2_ref[1, 0, :N_NODES] + p2_ref[1, 1, :N_NODES]], axis=1)
    h3 = jnp.maximum(agg2 + y2r2_ref[:, 64:] + b5_ref[...], 0.0)
    seg = jax.lax.broadcasted_iota(jnp.int32, (64, N_NODES), 0)
    mask = (seg == batch_ref[...][None, :]).astype(jnp.float32)
    pooled = mask @ h3
    z = jnp.maximum(pooled @ wfc1_ref[...] + bfc1_ref[...], 0.0)
    logits = z @ wfc2_ref[...] + bfc2_ref[...]
    out_ref[...] = jax.nn.sigmoid(logits)


def kernel(x, edge_index, batch, W_rel1, W_root1, b1, W_gat, att_src, att_dst,
           b_gat, W_rel5, W_root5, b5, W_fc1, b_fc1, W_fc2, b_fc2):
    n = x.shape[0]
    src = edge_index[0]
    dst = edge_index[1]

    xr, xroot = pl.pallas_call(
        _tc1_body,
        out_shape=[jax.ShapeDtypeStruct((1, n, 64), jnp.float32),
                   jax.ShapeDtypeStruct((n, 64), jnp.float32)],
    )(x, jnp.concatenate([W_rel1, W_root1], axis=1), b1)
    srcp, dstp, per = _pad_edges(src, dst, EB)
    parts = _seg_sum_rows(xr, srcp, dstp, per)
    h = pl.pallas_call(
        _tc2a_body,
        out_shape=jax.ShapeDtypeStruct((n, 64), jnp.float32),
    )(parts, xroot)

    loop = jnp.arange(n, dtype=src.dtype)
    sE, dE, perE = _pad_edges(jnp.concatenate([src, loop]),
                              jnp.concatenate([dst, loop]), EB)
    g_cm = pl.pallas_call(
        _tc2b_body,
        grid=(NCHUNK,),
        in_specs=[pl.BlockSpec((n, 64), lambda c: (0, 0)),
                  pl.BlockSpec((1, 64, CW), lambda c: (c, 0, 0))],
        out_specs=pl.BlockSpec((1, n, CW), lambda c: (c, 0, 0)),
        out_shape=jax.ShapeDtypeStruct((NCHUNK, n, CW), jnp.float32),
    )(h, jnp.moveaxis(W_gat.reshape(64, NCHUNK, CW), 1, 0))
    AS, BC = pl.pallas_call(
        _tc2c_body,
        out_shape=[jax.ShapeDtypeStruct((n, H), jnp.float32),
                   jax.ShapeDtypeStruct((N_PAD, 2 * H), jnp.float32)],
    )(h, W_gat, att_src, att_dst)
    w = _edge_softmax(AS, BC, sE, dE, perE)
    h2_cm = _gat_messages(g_cm, w, sE, dE, b_gat, perE * NC)

    y2r2 = pl.pallas_call(
        _tc4_body,
        grid=(NCHUNK,),
        in_specs=[pl.BlockSpec((1, N_PAD, CW), lambda c: (c, 0, 0)),
                  pl.BlockSpec((1, CW, 128), lambda c: (c, 0, 0))],
        out_specs=pl.BlockSpec((n, 128), lambda c: (0, 0)),
        out_shape=jax.ShapeDtypeStruct((n, 128), jnp.float32),
    )(h2_cm, jnp.concatenate([W_rel5, W_root5], axis=1).reshape(
        NCHUNK, CW, 128))
    y232 = jnp.stack([y2r2[:, :32], y2r2[:, 32:64]])
    parts2 = _seg_sum_rows(y232, srcp, dstp, per)
    return pl.pallas_call(
        _tc6_body,
        out_shape=jax.ShapeDtypeStruct((64, W_fc2.shape[1]), jnp.float32),
    )(parts2, y2r2, b5, batch, W_fc1, b_fc1, W_fc2, b_fc2)
